# Initial kernel scaffold; baseline (speedup 1.0000x reference)
import jax
import jax.numpy as jnp
from jax.experimental import pallas as pl


def kernel(x, edge_index, weight_neigh, weight_self):
    raise NotImplementedError("write your pallas kernel here")


# speedup vs baseline: 8.9412x; 8.9412x over previous
import jax
import jax.numpy as jnp
from jax import lax
from jax.experimental import pallas as pl
from jax.experimental.pallas import tpu as pltpu
from jax.experimental.pallas import tpu_sc as plsc

N = 10000
E = 320000
D = 128
L = 16
W = 16
NSC = 2
NTILES = 16
WORKERS = NSC * NTILES
EPW = E // WORKERS
WINDOWS = EPW // W
NPAD = 10240
RPT = NPAD // NTILES
ZROWS = 16
NBUF = 5


def _self_ip_body(x_ref, s_ref):
    xb = x_ref[...]
    sq = xb * xb
    s_ref[...] = 2.0 * sq[:, D - 1] - jnp.sum(sq, axis=1)


def _final_body(p0_ref, p1_ref, x_ref, wn_ref, ws_ref, o_ref):
    P = p0_ref[...] + p1_ref[...]
    wsum = jnp.maximum(P[:, D - 1 : D], 1e-6)
    nf = P / wsum
    xb = x_ref[...]
    acc = jnp.dot(nf, wn_ref[...], preferred_element_type=jnp.float32)
    acc = acc + jnp.dot(xb, ws_ref[...], preferred_element_type=jnp.float32)
    o_ref[...] = jnp.maximum(acc, 0.0)


def _sc_edge_kernel(x_hbm, pidx_hbm, s_hbm, out_hbm,
                    ibuf, arows, brows, obuf, aabuf, bbbuf,
                    gr, gc, sidx, semg, semsc, semi, zbuf, accum):
    cid = lax.axis_index("c")
    sid = lax.axis_index("s")
    wid = sid * NSC + cid

    zeros = jnp.zeros((L,), jnp.float32)

    def zrow(r, carry):
        for k in range(D // L):
            zbuf[r, pl.ds(k * L, L)] = zeros
        return carry

    lax.fori_loop(0, ZROWS, zrow, 0)
    for j in range(RPT // ZROWS):
        pltpu.sync_copy(zbuf, accum.at[pl.ds(sid * RPT + j * ZROWS, ZROWS)])

    base = wid * EPW
    plsc.subcore_barrier()

    lane = lax.iota(jnp.int32, L)
    metric = jnp.where(lane == L - 1, -1.0, 1.0).astype(jnp.float32)
    is_last = lane == L - 1

    def issue_idx(i, p):
        pltpu.async_copy(pidx_hbm.at[pl.ds(base + i * W, W)], ibuf[p], semi[p])

    def drain_idx(p):
        pltpu.make_async_copy(pidx_hbm.at[pl.ds(0, W)], ibuf[p], semi[p]).wait()

    def unpack_and_issue(p):
        pv = ibuf[p][pl.ds(0, L)]
        gr[p][pl.ds(0, L)] = jnp.bitwise_and(pv, 0xFFFF)
        gc[p][pl.ds(0, L)] = jnp.right_shift(pv, 16)
        pltpu.async_copy(x_hbm.at[gr[p]], arows[p], semg[p])
        pltpu.async_copy(x_hbm.at[gc[p]], brows[p], semg[p])
        pltpu.async_copy(s_hbm.at[gr[p]], aabuf[p], semg[p])
        pltpu.async_copy(s_hbm.at[gc[p]], bbbuf[p], semg[p])

    def drain_gathers(p):
        pltpu.make_async_copy(x_hbm.at[pl.ds(0, W)], arows[p], semg[p]).wait()
        pltpu.make_async_copy(x_hbm.at[pl.ds(0, W)], brows[p], semg[p]).wait()
        pltpu.make_async_copy(s_hbm.at[pl.ds(0, W)], aabuf[p], semg[p]).wait()
        pltpu.make_async_copy(s_hbm.at[pl.ds(0, W)], bbbuf[p], semg[p]).wait()

    def drain_scatter(p):
        pltpu.make_async_copy(x_hbm.at[pl.ds(0, W)], obuf[p], semsc[p]).wait()

    for p in range(NBUF):
        issue_idx(p, p)
    for p in range(NBUF):
        drain_idx(p)
        unpack_and_issue(p)

    def process(i, j, p):
        @pl.when(j < WINDOWS // NBUF - 1)
        def _():
            issue_idx(i + NBUF, p)

        drain_gathers(p)

        @pl.when(j >= 1)
        def _():
            drain_scatter(p)

        sidx[p][pl.ds(0, L)] = gr[p][pl.ds(0, L)]

        aav = aabuf[p][pl.ds(0, L)]
        bbv = bbbuf[p][pl.ds(0, L)]
        denv = aav * bbv
        dinv = 1.0 / (jnp.maximum(jnp.abs(denv), 1e-9) * jnp.sign(denv))
        for l in range(L):
            a = [arows[p][l, pl.ds(k * L, L)] for k in range(D // L)]
            b = [brows[p][l, pl.ds(k * L, L)] for k in range(D // L)]
            t = a[0] * b[0]
            for k in range(1, D // L - 1):
                t = t + a[k] * b[k]
            t = t + (a[D // L - 1] * b[D // L - 1]) * metric
            for k in (8, 4, 2, 1):
                t = t + jnp.take(t, (lane + k) % L)
            wv = jnp.exp((denv[l] - t * t) * dinv[l])
            for k in range(D // L - 1):
                obuf[p][l, pl.ds(k * L, L)] = wv * b[k]
            last = jnp.where(is_last, 1.0, b[D // L - 1])
            obuf[p][l, pl.ds((D // L - 1) * L, L)] = wv * last

        pltpu.async_copy(obuf[p], accum.at[sidx[p]], semsc[p], add=True)

        @pl.when(j < WINDOWS // NBUF - 1)
        def _():
            drain_idx(p)
            unpack_and_issue(p)

    def superstep(j, carry):
        for p in range(NBUF):
            process(j * NBUF + p, j, p)
        return carry

    lax.fori_loop(0, WINDOWS // NBUF, superstep, 0)
    for p in range(NBUF):
        drain_scatter(p)

    plsc.subcore_barrier()
    pltpu.sync_copy(accum.at[pl.ds(sid * RPT, RPT)],
                    out_hbm.at[cid, pl.ds(sid * RPT, RPT)])


_sc_edge = pl.kernel(
    _sc_edge_kernel,
    out_type=jax.ShapeDtypeStruct((NSC, NPAD, D), jnp.float32),
    mesh=plsc.VectorSubcoreMesh(
        core_axis_name="c", subcore_axis_name="s",
        num_cores=NSC, num_subcores=NTILES),
    scratch_types=[
        [pltpu.VMEM((W,), jnp.int32)] * NBUF,
        [pltpu.VMEM((W, D), jnp.float32)] * NBUF,
        [pltpu.VMEM((W, D), jnp.float32)] * NBUF,
        [pltpu.VMEM((W, D), jnp.float32)] * NBUF,
        [pltpu.VMEM((W,), jnp.float32)] * NBUF,
        [pltpu.VMEM((W,), jnp.float32)] * NBUF,
        [pltpu.VMEM((W,), jnp.int32)] * NBUF,
        [pltpu.VMEM((W,), jnp.int32)] * NBUF,
        [pltpu.VMEM((W,), jnp.int32)] * NBUF,
        [pltpu.SemaphoreType.DMA] * NBUF,
        [pltpu.SemaphoreType.DMA] * NBUF,
        [pltpu.SemaphoreType.DMA] * NBUF,
        pltpu.VMEM((ZROWS, D), jnp.float32),
        pltpu.VMEM_SHARED((NPAD, D), jnp.float32),
    ],
)


def kernel(x, edge_index, weight_neigh, weight_self):
    x = x.astype(jnp.float32)
    row = edge_index[0].astype(jnp.int32)
    col = edge_index[1].astype(jnp.int32)
    pidx = jnp.bitwise_or(row, jnp.left_shift(col, 16))

    s = pl.pallas_call(
        _self_ip_body,
        out_shape=jax.ShapeDtypeStruct((N,), jnp.float32),
    )(x)

    partials = _sc_edge(x, pidx, s)

    wn_pad = jnp.zeros((D, D), jnp.float32).at[: D - 1, : D - 1].set(
        weight_neigh.T.astype(jnp.float32))
    ws_pad = jnp.zeros((D, D), jnp.float32).at[: D - 1, : D - 1].set(
        weight_self.T.astype(jnp.float32)).at[D - 1, D - 1].set(1.0)

    BR = 1000
    out = pl.pallas_call(
        _final_body,
        grid=(N // BR,),
        in_specs=[
            pl.BlockSpec((BR, D), lambda i: (i, 0)),
            pl.BlockSpec((BR, D), lambda i: (i, 0)),
            pl.BlockSpec((BR, D), lambda i: (i, 0)),
            pl.BlockSpec((D, D), lambda i: (0, 0)),
            pl.BlockSpec((D, D), lambda i: (0, 0)),
        ],
        out_specs=pl.BlockSpec((BR, D), lambda i: (i, 0)),
        out_shape=jax.ShapeDtypeStruct((N, D), jnp.float32),
    )(partials[0], partials[1], x, wn_pad, ws_pad)
    return out


# speedup vs baseline: 8.9889x; 1.0053x over previous
import functools

import jax
import jax.numpy as jnp
from jax import lax
from jax.experimental import pallas as pl
from jax.experimental.pallas import tpu as pltpu
from jax.experimental.pallas import tpu_sc as plsc

N = 10000
E = 320000
D = 128
L = 16
W = 32
NSC = 2
NTILES = 16
WORKERS = NSC * NTILES
EPW = E // WORKERS
WINDOWS = EPW // W
NPAD = 10240
RPT = NPAD // NTILES
ZROWS = 16
NBUF = 2


def _self_ip_body(x_ref, s_ref):
    xb = x_ref[...]
    sq = xb * xb
    s_ref[...] = 2.0 * sq[:, D - 1] - jnp.sum(sq, axis=1)


def _final_body(p0_ref, p1_ref, x_ref, wn_ref, ws_ref, o_ref):
    P = p0_ref[...] + p1_ref[...]
    wsum = jnp.maximum(P[:, D - 1 : D], 1e-6)
    nf = P / wsum
    xb = x_ref[...]
    acc = jnp.dot(nf, wn_ref[...], preferred_element_type=jnp.float32)
    acc = acc + jnp.dot(xb, ws_ref[...], preferred_element_type=jnp.float32)
    o_ref[...] = jnp.maximum(acc, 0.0)


def _sc_edge_kernel(x_hbm, pidx_hbm, s_hbm, out_hbm,
                    pidx_all,
                    arows, brows, obuf, aabuf, bbbuf, gr, gc, sidx,
                    semg, semsc, tidx, zbuf, accum):
    cid = lax.axis_index("c")
    sid = lax.axis_index("s")
    wid = sid * NSC + cid

    zeros = jnp.zeros((L,), jnp.float32)

    def zrow(r, carry):
        for k in range(D // L):
            zbuf[r, pl.ds(k * L, L)] = zeros
        return carry

    lax.fori_loop(0, ZROWS, zrow, 0)
    for j in range(RPT // ZROWS):
        pltpu.sync_copy(zbuf, accum.at[pl.ds(sid * RPT + j * ZROWS, ZROWS)])

    base = wid * EPW
    pltpu.sync_copy(pidx_hbm.at[pl.ds(base, EPW)], pidx_all)
    plsc.subcore_barrier()

    lane = lax.iota(jnp.int32, L)
    metric = jnp.where(lane == L - 1, -1.0, 1.0).astype(jnp.float32)
    is_last = lane == L - 1

    def issue_gathers(i, p):
        for q in range(W // L):
            pv = pidx_all[pl.ds(i * W + q * L, L)]
            gr[p][pl.ds(q * L, L)] = jnp.bitwise_and(pv, 0xFFFF)
            gc[p][pl.ds(q * L, L)] = jnp.right_shift(pv, 16)
        pltpu.async_copy(x_hbm.at[gr[p]], arows[p], semg[p])
        pltpu.async_copy(x_hbm.at[gc[p]], brows[p], semg[p])
        pltpu.async_copy(s_hbm.at[gr[p]], aabuf[p], semg[p])
        pltpu.async_copy(s_hbm.at[gc[p]], bbbuf[p], semg[p])

    def drain_gathers(p):
        pltpu.make_async_copy(x_hbm.at[pl.ds(0, W)], arows[p], semg[p]).wait()
        pltpu.make_async_copy(x_hbm.at[pl.ds(0, W)], brows[p], semg[p]).wait()
        pltpu.make_async_copy(s_hbm.at[pl.ds(0, W)], aabuf[p], semg[p]).wait()
        pltpu.make_async_copy(s_hbm.at[pl.ds(0, W)], bbbuf[p], semg[p]).wait()

    def drain_scatter(p):
        pltpu.make_async_copy(x_hbm.at[pl.ds(0, W)], obuf[p], semsc[p]).wait()

    for p in range(NBUF):
        issue_gathers(p, p)

    def compute_edges(p, nq):
        for q in range(nq):
            aav = aabuf[p][pl.ds(q * L, L)]
            bbv = bbbuf[p][pl.ds(q * L, L)]
            denv = aav * bbv
            dinv = 1.0 / (jnp.maximum(jnp.abs(denv), 1e-9) * jnp.sign(denv))
            e0 = q * L
            for l in range(L):
                e = e0 + l
                a = [arows[p][e, pl.ds(k * L, L)] for k in range(D // L)]
                b = [brows[p][e, pl.ds(k * L, L)] for k in range(D // L)]
                t = a[0] * b[0]
                for k in range(1, D // L - 1):
                    t = t + a[k] * b[k]
                t = t + (a[D // L - 1] * b[D // L - 1]) * metric
                for k in (8, 4, 2, 1):
                    t = t + jnp.take(t, (lane + k) % L)
                wv = jnp.exp((denv[l] - t * t) * dinv[l])
                for k in range(D // L - 1):
                    obuf[p][e, pl.ds(k * L, L)] = wv * b[k]
                last = jnp.where(is_last, 1.0, b[D // L - 1])
                obuf[p][e, pl.ds((D // L - 1) * L, L)] = wv * last

    def process(i, p):
        drain_gathers(p)

        @pl.when(i >= NBUF)
        def _():
            drain_scatter(p)

        for q in range(W // L):
            sidx[p][pl.ds(q * L, L)] = gr[p][pl.ds(q * L, L)]

        compute_edges(p, W // L)
        pltpu.async_copy(obuf[p], accum.at[sidx[p]], semsc[p], add=True)

        @pl.when(i + NBUF < WINDOWS)
        def _():
            issue_gathers(i + NBUF, p)

    def window(i, carry):
        for p in range(NBUF):
            @pl.when(lax.rem(i, NBUF) == p)
            def _(p=p):
                process(i, p)

        return carry

    lax.fori_loop(0, WINDOWS, window, 0)
    for j in range(NBUF):
        drain_scatter((WINDOWS - NBUF + j) % NBUF)

    toff = WINDOWS * W
    pv = pidx_all[pl.ds(toff, L)]
    gr[0][pl.ds(0, L)] = jnp.bitwise_and(pv, 0xFFFF)
    gc[0][pl.ds(0, L)] = jnp.right_shift(pv, 16)
    tidx[pl.ds(0, L)] = jnp.bitwise_and(pv, 0xFFFF)
    pltpu.async_copy(x_hbm.at[gr[0].at[pl.ds(0, L)]], arows[0].at[pl.ds(0, L)], semg[0])
    pltpu.async_copy(x_hbm.at[gc[0].at[pl.ds(0, L)]], brows[0].at[pl.ds(0, L)], semg[0])
    pltpu.async_copy(s_hbm.at[gr[0].at[pl.ds(0, L)]], aabuf[0].at[pl.ds(0, L)], semg[0])
    pltpu.async_copy(s_hbm.at[gc[0].at[pl.ds(0, L)]], bbbuf[0].at[pl.ds(0, L)], semg[0])
    pltpu.make_async_copy(x_hbm.at[pl.ds(0, L)], arows[0].at[pl.ds(0, L)], semg[0]).wait()
    pltpu.make_async_copy(x_hbm.at[pl.ds(0, L)], brows[0].at[pl.ds(0, L)], semg[0]).wait()
    pltpu.make_async_copy(s_hbm.at[pl.ds(0, L)], aabuf[0].at[pl.ds(0, L)], semg[0]).wait()
    pltpu.make_async_copy(s_hbm.at[pl.ds(0, L)], bbbuf[0].at[pl.ds(0, L)], semg[0]).wait()
    compute_edges(0, 1)
    pltpu.sync_copy(obuf[0].at[pl.ds(0, L)], accum.at[tidx], add=True)

    plsc.subcore_barrier()
    pltpu.sync_copy(accum.at[pl.ds(sid * RPT, RPT)],
                    out_hbm.at[cid, pl.ds(sid * RPT, RPT)])


_sc_edge = pl.kernel(
    _sc_edge_kernel,
    out_type=jax.ShapeDtypeStruct((NSC, NPAD, D), jnp.float32),
    mesh=plsc.VectorSubcoreMesh(
        core_axis_name="c", subcore_axis_name="s",
        num_cores=NSC, num_subcores=NTILES),
    scratch_types=[
        pltpu.VMEM((EPW,), jnp.int32),
        [pltpu.VMEM((W, D), jnp.float32)] * NBUF,
        [pltpu.VMEM((W, D), jnp.float32)] * NBUF,
        [pltpu.VMEM((W, D), jnp.float32)] * NBUF,
        [pltpu.VMEM((W,), jnp.float32)] * NBUF,
        [pltpu.VMEM((W,), jnp.float32)] * NBUF,
        [pltpu.VMEM((W,), jnp.int32)] * NBUF,
        [pltpu.VMEM((W,), jnp.int32)] * NBUF,
        [pltpu.VMEM((W,), jnp.int32)] * NBUF,
        [pltpu.SemaphoreType.DMA] * NBUF,
        [pltpu.SemaphoreType.DMA] * NBUF,
        pltpu.VMEM((L,), jnp.int32),
        pltpu.VMEM((ZROWS, D), jnp.float32),
        pltpu.VMEM_SHARED((NPAD, D), jnp.float32),
    ],
)


def kernel(x, edge_index, weight_neigh, weight_self):
    x = x.astype(jnp.float32)
    row = edge_index[0].astype(jnp.int32)
    col = edge_index[1].astype(jnp.int32)
    pidx = jnp.bitwise_or(row, jnp.left_shift(col, 16))

    BR = 1000
    s = pl.pallas_call(
        _self_ip_body,
        out_shape=jax.ShapeDtypeStruct((N,), jnp.float32),
    )(x)

    partials = _sc_edge(x, pidx, s)

    wn_pad = jnp.zeros((D, D), jnp.float32).at[: D - 1, : D - 1].set(
        weight_neigh.T.astype(jnp.float32))
    ws_pad = jnp.zeros((D, D), jnp.float32).at[: D - 1, : D - 1].set(
        weight_self.T.astype(jnp.float32)).at[D - 1, D - 1].set(1.0)

    out = pl.pallas_call(
        _final_body,
        grid=(N // BR,),
        in_specs=[
            pl.BlockSpec((BR, D), lambda i: (i, 0)),
            pl.BlockSpec((BR, D), lambda i: (i, 0)),
            pl.BlockSpec((BR, D), lambda i: (i, 0)),
            pl.BlockSpec((D, D), lambda i: (0, 0)),
            pl.BlockSpec((D, D), lambda i: (0, 0)),
        ],
        out_specs=pl.BlockSpec((BR, D), lambda i: (i, 0)),
        out_shape=jax.ShapeDtypeStruct((N, D), jnp.float32),
    )(partials[0], partials[1], x, wn_pad, ws_pad)
    return out


# speedup vs baseline: 9.6847x; 1.0774x over previous
import functools

import jax
import jax.numpy as jnp
from jax import lax
from jax.experimental import pallas as pl
from jax.experimental.pallas import tpu as pltpu
from jax.experimental.pallas import tpu_sc as plsc

N = 10000
E = 320000
D = 128
L = 16
W = 16
NSC = 2
NTILES = 16
WORKERS = NSC * NTILES
EPW = E // WORKERS
WINDOWS = EPW // W
NPAD = 10240
RPT = NPAD // NTILES
ZROWS = 16
NBUF = 4


def _self_ip_body(x_ref, s_ref):
    xb = x_ref[...]
    sq = xb * xb
    s_ref[...] = 2.0 * sq[:, D - 1] - jnp.sum(sq, axis=1)


def _final_body(p0_ref, p1_ref, x_ref, wn_ref, ws_ref, o_ref):
    P = p0_ref[...] + p1_ref[...]
    wsum = jnp.maximum(P[:, D - 1 : D], 1e-6)
    nf = P / wsum
    xb = x_ref[...]
    acc = jnp.dot(nf, wn_ref[...], preferred_element_type=jnp.float32)
    acc = acc + jnp.dot(xb, ws_ref[...], preferred_element_type=jnp.float32)
    o_ref[...] = jnp.maximum(acc, 0.0)


def _sc_edge_kernel(x_hbm, pidx_hbm, s_hbm, out_hbm,
                    pidx_all,
                    arows, brows, obuf, aabuf, bbbuf, gr, gc, sidx,
                    semg, semsc, zbuf, accum):
    cid = lax.axis_index("c")
    sid = lax.axis_index("s")
    wid = sid * NSC + cid

    zeros = jnp.zeros((L,), jnp.float32)

    def zrow(r, carry):
        for k in range(D // L):
            zbuf[r, pl.ds(k * L, L)] = zeros
        return carry

    lax.fori_loop(0, ZROWS, zrow, 0)
    for j in range(RPT // ZROWS):
        pltpu.sync_copy(zbuf, accum.at[pl.ds(sid * RPT + j * ZROWS, ZROWS)])

    base = wid * EPW
    pltpu.sync_copy(pidx_hbm.at[pl.ds(base, EPW)], pidx_all)
    plsc.subcore_barrier()

    lane = lax.iota(jnp.int32, L)
    metric = jnp.where(lane == L - 1, -1.0, 1.0).astype(jnp.float32)
    is_last = lane == L - 1

    def issue_gathers(i, p):
        for q in range(W // L):
            pv = pidx_all[pl.ds(i * W + q * L, L)]
            gr[p][pl.ds(q * L, L)] = jnp.bitwise_and(pv, 0xFFFF)
            gc[p][pl.ds(q * L, L)] = jnp.right_shift(pv, 16)
        pltpu.async_copy(x_hbm.at[gr[p]], arows[p], semg[p])
        pltpu.async_copy(x_hbm.at[gc[p]], brows[p], semg[p])
        pltpu.async_copy(s_hbm.at[gr[p]], aabuf[p], semg[p])
        pltpu.async_copy(s_hbm.at[gc[p]], bbbuf[p], semg[p])

    def drain_gathers(p):
        pltpu.make_async_copy(x_hbm.at[pl.ds(0, W)], arows[p], semg[p]).wait()
        pltpu.make_async_copy(x_hbm.at[pl.ds(0, W)], brows[p], semg[p]).wait()
        pltpu.make_async_copy(s_hbm.at[pl.ds(0, W)], aabuf[p], semg[p]).wait()
        pltpu.make_async_copy(s_hbm.at[pl.ds(0, W)], bbbuf[p], semg[p]).wait()

    def drain_scatter(p):
        pltpu.make_async_copy(x_hbm.at[pl.ds(0, W)], obuf[p], semsc[p]).wait()

    for p in range(NBUF):
        issue_gathers(p, p)

    def process(i, p):
        drain_gathers(p)

        @pl.when(i >= NBUF)
        def _():
            drain_scatter(p)

        for q in range(W // L):
            sidx[p][pl.ds(q * L, L)] = gr[p][pl.ds(q * L, L)]

        def qbody(q, qcarry):
            aav = aabuf[p][pl.ds(q * L, L)]
            bbv = bbbuf[p][pl.ds(q * L, L)]
            denv = aav * bbv
            dinv = 1.0 / (jnp.maximum(jnp.abs(denv), 1e-9) * jnp.sign(denv))
            e0 = q * L
            for l in range(L):
                e = e0 + l
                a = [arows[p][e, pl.ds(k * L, L)] for k in range(D // L)]
                b = [brows[p][e, pl.ds(k * L, L)] for k in range(D // L)]
                t = a[0] * b[0]
                for k in range(1, D // L - 1):
                    t = t + a[k] * b[k]
                t = t + (a[D // L - 1] * b[D // L - 1]) * metric
                for k in (8, 4, 2, 1):
                    t = t + jnp.take(t, (lane + k) % L)
                wv = jnp.exp((denv[l] - t * t) * dinv[l])
                for k in range(D // L - 1):
                    obuf[p][e, pl.ds(k * L, L)] = wv * b[k]
                last = jnp.where(is_last, 1.0, b[D // L - 1])
                obuf[p][e, pl.ds((D // L - 1) * L, L)] = wv * last
            return qcarry

        lax.fori_loop(0, W // L, qbody, 0)
        pltpu.async_copy(obuf[p], accum.at[sidx[p]], semsc[p], add=True)

        @pl.when(i + NBUF < WINDOWS)
        def _():
            issue_gathers(i + NBUF, p)

    def window(i, carry):
        for p in range(NBUF):
            @pl.when(lax.rem(i, NBUF) == p)
            def _(p=p):
                process(i, p)

        return carry

    lax.fori_loop(0, WINDOWS, window, 0)
    for j in range(NBUF):
        drain_scatter((WINDOWS - NBUF + j) % NBUF)

    plsc.subcore_barrier()
    pltpu.sync_copy(accum.at[pl.ds(sid * RPT, RPT)],
                    out_hbm.at[cid, pl.ds(sid * RPT, RPT)])


_sc_edge = pl.kernel(
    _sc_edge_kernel,
    out_type=jax.ShapeDtypeStruct((NSC, NPAD, D), jnp.float32),
    mesh=plsc.VectorSubcoreMesh(
        core_axis_name="c", subcore_axis_name="s",
        num_cores=NSC, num_subcores=NTILES),
    scratch_types=[
        pltpu.VMEM((EPW,), jnp.int32),
        [pltpu.VMEM((W, D), jnp.float32)] * NBUF,
        [pltpu.VMEM((W, D), jnp.float32)] * NBUF,
        [pltpu.VMEM((W, D), jnp.float32)] * NBUF,
        [pltpu.VMEM((W,), jnp.float32)] * NBUF,
        [pltpu.VMEM((W,), jnp.float32)] * NBUF,
        [pltpu.VMEM((W,), jnp.int32)] * NBUF,
        [pltpu.VMEM((W,), jnp.int32)] * NBUF,
        [pltpu.VMEM((W,), jnp.int32)] * NBUF,
        [pltpu.SemaphoreType.DMA] * NBUF,
        [pltpu.SemaphoreType.DMA] * NBUF,
        pltpu.VMEM((ZROWS, D), jnp.float32),
        pltpu.VMEM_SHARED((NPAD, D), jnp.float32),
    ],
)


def kernel(x, edge_index, weight_neigh, weight_self):
    x = x.astype(jnp.float32)
    row = edge_index[0].astype(jnp.int32)
    col = edge_index[1].astype(jnp.int32)
    pidx = jnp.bitwise_or(row, jnp.left_shift(col, 16))

    BR = 1000
    s = pl.pallas_call(
        _self_ip_body,
        out_shape=jax.ShapeDtypeStruct((N,), jnp.float32),
    )(x)

    partials = _sc_edge(x, pidx, s)

    wn_pad = jnp.zeros((D, D), jnp.float32).at[: D - 1, : D - 1].set(
        weight_neigh.T.astype(jnp.float32))
    ws_pad = jnp.zeros((D, D), jnp.float32).at[: D - 1, : D - 1].set(
        weight_self.T.astype(jnp.float32)).at[D - 1, D - 1].set(1.0)

    out = pl.pallas_call(
        _final_body,
        grid=(N // BR,),
        in_specs=[
            pl.BlockSpec((BR, D), lambda i: (i, 0)),
            pl.BlockSpec((BR, D), lambda i: (i, 0)),
            pl.BlockSpec((BR, D), lambda i: (i, 0)),
            pl.BlockSpec((D, D), lambda i: (0, 0)),
            pl.BlockSpec((D, D), lambda i: (0, 0)),
        ],
        out_specs=pl.BlockSpec((BR, D), lambda i: (i, 0)),
        out_shape=jax.ShapeDtypeStruct((N, D), jnp.float32),
    )(partials[0], partials[1], x, wn_pad, ws_pad)
    return out
